# baseline (speedup 1.0000x reference)
import functools

import jax
import jax.numpy as jnp
from jax import lax
from jax.experimental import pallas as pl
from jax.experimental.pallas import tpu as pltpu
from jax.experimental.pallas import tpu_sc as plsc

_NC = 2
_NS = 16
_NW = _NC * _NS


def _make_pool(B, L, V, D):
    bpw = B // _NW
    blk = 32
    nblk = bpw // blk

    mesh = plsc.VectorSubcoreMesh(core_axis_name="c", subcore_axis_name="s")

    @functools.partial(
        pl.kernel,
        mesh=mesh,
        out_type=jax.ShapeDtypeStruct((B, D), jnp.float32),
        scratch_types=[
            pltpu.VMEM((L, bpw), jnp.int32),
            pltpu.VMEM((bpw, D), jnp.float32),
        ] + [pltpu.SemaphoreType.DMA] * 16,
    )
    def pool(ids_hbm, table_hbm, out_hbm, idx_v, acc_v, *sems):
        cid = lax.axis_index("c")
        sid = lax.axis_index("s")
        wid = sid * _NC + cid
        accs = tuple(acc_v.at[pl.ds(k * blk, blk)] for k in range(nblk))

        pltpu.sync_copy(ids_hbm.at[wid], idx_v)

        for k in range(nblk):
            pltpu.make_async_copy(
                table_hbm.at[idx_v.at[0, pl.ds(k * blk, blk)]], accs[k], sems[k]).start()

        def step(t, carry):
            for k in range(nblk):
                pltpu.make_async_copy(
                    table_hbm.at[idx_v.at[t - 1, pl.ds(k * blk, blk)]], accs[k],
                    sems[k]).wait()
                pltpu.async_copy(
                    table_hbm.at[idx_v.at[t, pl.ds(k * blk, blk)]], accs[k], sems[k],
                    add=True)
            return carry

        lax.fori_loop(1, L, step, 0)

        for k in range(nblk):
            pltpu.make_async_copy(
                table_hbm.at[idx_v.at[L - 1, pl.ds(k * blk, blk)]], accs[k], sems[k]).wait()
        pltpu.sync_copy(acc_v, out_hbm.at[pl.ds(wid * bpw, bpw)])

    return pool


def _mm_body(x_ref, w_ref, b_ref, o_ref):
    o_ref[...] = jnp.dot(
        x_ref[...], w_ref[...], preferred_element_type=jnp.float32
    ) + b_ref[...]


@jax.jit
def kernel(input_ids, emb_table, fc_w, fc_b):
    B, L = input_ids.shape
    V, D = emb_table.shape
    O = fc_w.shape[1]
    bpw = B // _NW
    blk = 32
    nblk = bpw // blk

    ids = (input_ids.astype(jnp.int32)
           .reshape(_NW, bpw, L)
           .transpose(0, 2, 1))

    pool = _make_pool(B, L, V, D)
    pooled = pool(ids, emb_table)

    w_scaled = fc_w * (1.0 / L)

    bm = 2048
    out = pl.pallas_call(
        _mm_body,
        grid=(B // bm,),
        in_specs=[
            pl.BlockSpec((bm, D), lambda i: (i, 0)),
            pl.BlockSpec((D, O), lambda i: (0, 0)),
            pl.BlockSpec((1, O), lambda i: (0, 0)),
        ],
        out_specs=pl.BlockSpec((bm, O), lambda i: (i, 0)),
        out_shape=jax.ShapeDtypeStruct((B, O), jnp.float32),
    )(pooled, w_scaled, fc_b.reshape(1, O))
    return out


# speedup vs baseline: 1.1088x; 1.1088x over previous
import functools

import jax
import jax.numpy as jnp
from jax import lax
from jax.experimental import pallas as pl
from jax.experimental.pallas import tpu as pltpu
from jax.experimental.pallas import tpu_sc as plsc

_NC = 2
_NS = 16
_NW = _NC * _NS


def _make_pool(B, L, V, D):
    bpw = B // _NW
    blk = 64
    nblk = bpw // blk

    mesh = plsc.VectorSubcoreMesh(core_axis_name="c", subcore_axis_name="s")

    @functools.partial(
        pl.kernel,
        mesh=mesh,
        out_type=jax.ShapeDtypeStruct((B, D), jnp.float32),
        scratch_types=[
            pltpu.VMEM((L, bpw), jnp.int32),
            pltpu.VMEM((bpw, D), jnp.float32),
        ] + [pltpu.SemaphoreType.DMA] * 8,
    )
    def pool(ids_hbm, table_hbm, out_hbm, idx_v, acc_v, *sems):
        cid = lax.axis_index("c")
        sid = lax.axis_index("s")
        wid = sid * _NC + cid
        accs = tuple(acc_v.at[pl.ds(k * blk, blk)] for k in range(nblk))

        pltpu.sync_copy(ids_hbm.at[wid], idx_v)

        for k in range(nblk):
            pltpu.make_async_copy(
                table_hbm.at[idx_v.at[0, pl.ds(k * blk, blk)]], accs[k], sems[k]).start()

        def step(t, carry):
            for k in range(nblk):
                pltpu.make_async_copy(
                    table_hbm.at[idx_v.at[t - 1, pl.ds(k * blk, blk)]], accs[k],
                    sems[k]).wait()
                pltpu.async_copy(
                    table_hbm.at[idx_v.at[t, pl.ds(k * blk, blk)]], accs[k], sems[k],
                    add=True)
            return carry

        lax.fori_loop(1, L, step, 0)

        for k in range(nblk):
            pltpu.make_async_copy(
                table_hbm.at[idx_v.at[L - 1, pl.ds(k * blk, blk)]], accs[k], sems[k]).wait()
        pltpu.sync_copy(acc_v, out_hbm.at[pl.ds(wid * bpw, bpw)])

    return pool


def _mm_body(x_ref, w_ref, b_ref, o_ref):
    o_ref[...] = jnp.dot(
        x_ref[...], w_ref[...], preferred_element_type=jnp.float32
    ) + b_ref[...]


@jax.jit
def kernel(input_ids, emb_table, fc_w, fc_b):
    B, L = input_ids.shape
    V, D = emb_table.shape
    O = fc_w.shape[1]
    bpw = B // _NW
    blk = 64
    nblk = bpw // blk

    ids = (input_ids.astype(jnp.int32)
           .reshape(_NW, bpw, L)
           .transpose(0, 2, 1))

    pool = _make_pool(B, L, V, D)
    pooled = pool(ids, emb_table)

    w_scaled = fc_w * (1.0 / L)

    bm = 2048
    out = pl.pallas_call(
        _mm_body,
        grid=(B // bm,),
        in_specs=[
            pl.BlockSpec((bm, D), lambda i: (i, 0)),
            pl.BlockSpec((D, O), lambda i: (0, 0)),
            pl.BlockSpec((1, O), lambda i: (0, 0)),
        ],
        out_specs=pl.BlockSpec((bm, O), lambda i: (i, 0)),
        out_shape=jax.ShapeDtypeStruct((B, O), jnp.float32),
    )(pooled, w_scaled, fc_b.reshape(1, O))
    return out
